# baseline (speedup 1.0000x reference)
import functools

import jax
import jax.numpy as jnp
from jax import lax
from jax.experimental import pallas as pl
from jax.experimental.pallas import tpu as pltpu
from jax.experimental.pallas import tpu_sc as plsc

B, D, H, E = 4096, 768, 1024, 8
BLK = 256
NBLK = B // BLK + E
PADB = NBLK * BLK
NC, NS = 2, 16
NW = NC * NS
CHUNK = B // NW
NGRP = CHUNK // 16

_mesh = plsc.VectorSubcoreMesh(core_axis_name="c", subcore_axis_name="s")


def _take16(x, idx):
    return lax.gather(
        x, idx[:, None],
        lax.GatherDimensionNumbers(offset_dims=(),
                                   collapsed_slice_dims=(0,),
                                   start_index_map=(0,)),
        slice_sizes=(1,),
        mode=lax.GatherScatterMode.PROMISE_IN_BOUNDS)


def _bcast_lane(x, lanes, i):
    return _take16(x, lanes * 0 + i)


def _cumsum16(x, lanes):
    for sh in (1, 2, 4, 8):
        idx = jnp.maximum(lanes - sh, 0)
        shifted = _take16(x, idx)
        x = x + jnp.where(lanes >= sh, shifted, 0)
    return x


@functools.partial(
    pl.kernel,
    mesh=_mesh,
    compiler_params=pltpu.CompilerParams(needs_layout_passes=False),
    out_type=(
        jax.ShapeDtypeStruct((PADB, D), jnp.float32),
        jax.ShapeDtypeStruct((B,), jnp.int32),
        jax.ShapeDtypeStruct((32,), jnp.int32),
    ),
    scratch_types=[
        pltpu.VMEM((B,), jnp.int32),
        pltpu.VMEM((CHUNK, D), jnp.float32),
        pltpu.VMEM((CHUNK,), jnp.int32),
        pltpu.VMEM((32,), jnp.int32),
        pltpu.VMEM((16,), jnp.int32),
        pltpu.VMEM((16,), jnp.int32),
        pltpu.VMEM((16,), jnp.int32),
        pltpu.SemaphoreType.DMA,
    ],
)
def _route(feat_hbm, aff_hbm, sorted_hbm, pos_hbm, bh_hbm,
           e_all, rows, pos_own, bh_v, base_v, hist_v, bef_v, sem):
    cid = lax.axis_index("c")
    sid = lax.axis_index("s")
    wid = sid * NC + cid
    base_tok = wid * CHUNK
    lanes = lax.broadcasted_iota(jnp.int32, (16,), 0)

    pltpu.sync_copy(aff_hbm, e_all)

    rows_cp = pltpu.async_copy(feat_hbm.at[pl.ds(base_tok, CHUNK)], rows, sem)

    zero16 = jnp.zeros((16,), jnp.int32)
    ones16 = zero16 + 1
    true16 = lanes >= 0
    hist_v[...] = zero16
    bef_v[...] = zero16

    def hist_body(i, carry):
        ev = e_all[pl.ds(i * 16, 16)]
        plsc.addupdate_scatter(hist_v, [ev], ones16)
        plsc.addupdate_scatter(bef_v, [ev], ones16,
                               mask=true16 & (i * 16 < base_tok))
        return carry

    lax.fori_loop(0, B // 16, hist_body, 0)
    cnt = hist_v[...]
    before = bef_v[...]

    padded = ((cnt + (BLK - 1)) // BLK) * BLK
    offs = _cumsum16(padded, lanes) - padded
    base_v[...] = offs + before

    counter = zero16
    for g in range(NGRP):
        ev = e_all[pl.ds(base_tok + g * 16, 16)]
        myb = plsc.load_gather(base_v, [ev])
        rank = zero16
        gh = zero16
        for h in range(E):
            m = ev == h
            cm = _cumsum16(jnp.where(m, 1, 0), lanes)
            tot = _bcast_lane(cm, lanes, 15)
            ch = _bcast_lane(counter, lanes, h)
            rank = jnp.where(m, cm - 1 + ch, rank)
            gh = gh + jnp.where(lanes == h, tot, 0)
        pos_own[pl.ds(g * 16, 16)] = myb + rank
        counter = counter + gh

    pltpu.sync_copy(pos_own, pos_hbm.at[pl.ds(base_tok, CHUNK)])

    rows_cp.wait()
    pltpu.async_copy(rows, sorted_hbm.at[pos_own], sem).wait()

    @pl.when(wid == 0)
    def _():
        blk_start = offs // BLK
        for half in range(2):
            jv = lanes + half * 16
            bcount = zero16
            for h in range(E):
                st = _bcast_lane(blk_start, lanes, h)
                bcount = bcount + jnp.where(st <= jv, 1, 0)
            bh_v[pl.ds(half * 16, 16)] = bcount - 1
        pltpu.sync_copy(bh_v, bh_hbm)


def _mlp_body(bh_ref, b2_ref, x_ref, w1_ref, b1_ref, w2_ref, o_ref):
    x = x_ref[...].astype(jnp.bfloat16)
    hpre = jnp.dot(x, w1_ref[0], preferred_element_type=jnp.float32)
    hp = hpre + b1_ref[0]
    hh = 0.5 * hp * (1.0 + lax.erf(hp * 0.7071067811865476))
    acc = jnp.sum(hh * w2_ref[0], axis=1)
    j = pl.program_id(0)
    o_ref[...] = (acc + b2_ref[bh_ref[j]]).reshape(1, 1, BLK)


_mlp = pl.pallas_call(
    _mlp_body,
    grid_spec=pltpu.PrefetchScalarGridSpec(
        num_scalar_prefetch=2,
        grid=(NBLK,),
        in_specs=[
            pl.BlockSpec((BLK, D), lambda j, bh, b2: (j, 0)),
            pl.BlockSpec((1, D, H), lambda j, bh, b2: (bh[j], 0, 0)),
            pl.BlockSpec((1, 1, H), lambda j, bh, b2: (bh[j], 0, 0)),
            pl.BlockSpec((1, 1, H), lambda j, bh, b2: (bh[j], 0, 0)),
        ],
        out_specs=pl.BlockSpec((1, 1, BLK), lambda j, bh, b2: (j, 0, 0)),
    ),
    out_shape=jax.ShapeDtypeStruct((NBLK, 1, BLK), jnp.float32),
)


@functools.partial(
    pl.kernel,
    mesh=_mesh,
    compiler_params=pltpu.CompilerParams(needs_layout_passes=False),
    out_type=jax.ShapeDtypeStruct((B,), jnp.float32),
    scratch_types=[
        pltpu.VMEM((PADB,), jnp.float32),
        pltpu.VMEM((CHUNK,), jnp.int32),
        pltpu.VMEM((CHUNK,), jnp.float32),
    ],
)
def _unpermute(pred_hbm, pos_hbm, out_hbm, pred_v, pos_v, out_v):
    cid = lax.axis_index("c")
    sid = lax.axis_index("s")
    wid = sid * NC + cid
    base = wid * CHUNK
    pltpu.sync_copy(pred_hbm, pred_v)
    pltpu.sync_copy(pos_hbm.at[pl.ds(base, CHUNK)], pos_v)
    for g in range(NGRP):
        idx = pos_v[pl.ds(g * 16, 16)]
        out_v[pl.ds(g * 16, 16)] = plsc.load_gather(pred_v, [idx])
    pltpu.sync_copy(out_v, out_hbm.at[pl.ds(base, CHUNK)])


@jax.jit
def kernel(features, affinity_type_idx, W1, b1, W2, b2):
    aff = affinity_type_idx.astype(jnp.int32)
    sorted_feat, pos, bh = _route(features, aff)
    pred = _mlp(bh, b2[:, 0], sorted_feat, W1.astype(jnp.bfloat16),
                b1[:, None, :], jnp.swapaxes(W2, 1, 2))
    return _unpermute(pred.reshape(PADB), pos)


# speedup vs baseline: 1.0777x; 1.0777x over previous
import functools

import jax
import jax.numpy as jnp
from jax import lax
from jax.experimental import pallas as pl
from jax.experimental.pallas import tpu as pltpu
from jax.experimental.pallas import tpu_sc as plsc

B, D, H, E = 4096, 768, 1024, 8
BLK = 256
NBLK = B // BLK + E
PADB = NBLK * BLK
NC, NS = 2, 16
NW = NC * NS
CHUNK = B // NW
NGRP = CHUNK // 16

_mesh = plsc.VectorSubcoreMesh(core_axis_name="c", subcore_axis_name="s")


def _take16(x, idx):
    return lax.gather(
        x, idx[:, None],
        lax.GatherDimensionNumbers(offset_dims=(),
                                   collapsed_slice_dims=(0,),
                                   start_index_map=(0,)),
        slice_sizes=(1,),
        mode=lax.GatherScatterMode.PROMISE_IN_BOUNDS)


def _bcast_lane(x, lanes, i):
    return _take16(x, lanes * 0 + i)


def _cumsum16(x, lanes):
    for sh in (1, 2, 4, 8):
        idx = jnp.maximum(lanes - sh, 0)
        shifted = _take16(x, idx)
        x = x + jnp.where(lanes >= sh, shifted, 0)
    return x


@functools.partial(
    pl.kernel,
    mesh=_mesh,
    compiler_params=pltpu.CompilerParams(needs_layout_passes=False),
    out_type=(
        jax.ShapeDtypeStruct((PADB, D), jnp.float32),
        jax.ShapeDtypeStruct((B,), jnp.int32),
        jax.ShapeDtypeStruct((32,), jnp.int32),
    ),
    scratch_types=[
        pltpu.VMEM((B,), jnp.int32),
        pltpu.VMEM((CHUNK, D), jnp.float32),
        pltpu.VMEM((CHUNK,), jnp.int32),
        pltpu.VMEM((32,), jnp.int32),
        pltpu.VMEM((16,), jnp.int32),
        pltpu.VMEM((16,), jnp.int32),
        pltpu.VMEM((16,), jnp.int32),
        pltpu.SemaphoreType.DMA,
    ],
)
def _route(feat_hbm, aff_hbm, sorted_hbm, pos_hbm, bh_hbm,
           e_all, rows, pos_own, bh_v, base_v, hist_v, bef_v, sem):
    cid = lax.axis_index("c")
    sid = lax.axis_index("s")
    wid = sid * NC + cid
    base_tok = wid * CHUNK
    lanes = lax.broadcasted_iota(jnp.int32, (16,), 0)

    pltpu.sync_copy(aff_hbm, e_all)

    rows_cp = pltpu.async_copy(feat_hbm.at[pl.ds(base_tok, CHUNK)], rows, sem)

    zero16 = jnp.zeros((16,), jnp.int32)
    ones16 = zero16 + 1
    true16 = lanes >= 0
    hist_v[...] = zero16
    bef_v[...] = zero16

    def hist_body(i, carry):
        ev = e_all[pl.ds(i * 16, 16)]
        plsc.addupdate_scatter(hist_v, [ev], ones16)
        plsc.addupdate_scatter(bef_v, [ev], ones16,
                               mask=true16 & (i * 16 < base_tok))
        return carry

    lax.fori_loop(0, B // 16, hist_body, 0)
    cnt = hist_v[...]
    before = bef_v[...]

    padded = ((cnt + (BLK - 1)) // BLK) * BLK
    offs = _cumsum16(padded, lanes) - padded
    base_v[...] = offs + before

    counter = zero16
    for g in range(NGRP):
        ev = e_all[pl.ds(base_tok + g * 16, 16)]
        myb = plsc.load_gather(base_v, [ev])
        rank = zero16
        gh = zero16
        for h in range(E):
            m = ev == h
            cm = _cumsum16(jnp.where(m, 1, 0), lanes)
            tot = _bcast_lane(cm, lanes, 15)
            ch = _bcast_lane(counter, lanes, h)
            rank = jnp.where(m, cm - 1 + ch, rank)
            gh = gh + jnp.where(lanes == h, tot, 0)
        pos_own[pl.ds(g * 16, 16)] = myb + rank
        counter = counter + gh

    pltpu.sync_copy(pos_own, pos_hbm.at[pl.ds(base_tok, CHUNK)])

    rows_cp.wait()
    pltpu.async_copy(rows, sorted_hbm.at[pos_own], sem).wait()

    @pl.when(wid == 0)
    def _():
        blk_start = offs // BLK
        for half in range(2):
            jv = lanes + half * 16
            bcount = zero16
            for h in range(E):
                st = _bcast_lane(blk_start, lanes, h)
                bcount = bcount + jnp.where(st <= jv, 1, 0)
            bh_v[pl.ds(half * 16, 16)] = bcount - 1
        pltpu.sync_copy(bh_v, bh_hbm)


def _mlp_body(bh_ref, b2_ref, x_ref, w1_ref, b1_ref, w2_ref, o_ref):
    x = x_ref[...]
    hpre = jnp.dot(x, w1_ref[0], preferred_element_type=jnp.float32)
    hp = hpre + b1_ref[0]
    hh = 0.5 * hp * (1.0 + lax.erf(hp * 0.7071067811865476))
    acc = jnp.sum(hh * w2_ref[0], axis=1)
    j = pl.program_id(0)
    o_ref[...] = (acc + b2_ref[bh_ref[j]]).reshape(1, 1, BLK)


_mlp = pl.pallas_call(
    _mlp_body,
    grid_spec=pltpu.PrefetchScalarGridSpec(
        num_scalar_prefetch=2,
        grid=(NBLK,),
        in_specs=[
            pl.BlockSpec((BLK, D), lambda j, bh, b2: (j, 0)),
            pl.BlockSpec((1, D, H), lambda j, bh, b2: (bh[j], 0, 0)),
            pl.BlockSpec((1, 1, H), lambda j, bh, b2: (bh[j], 0, 0)),
            pl.BlockSpec((1, 1, H), lambda j, bh, b2: (bh[j], 0, 0)),
        ],
        out_specs=pl.BlockSpec((1, 1, BLK), lambda j, bh, b2: (j, 0, 0)),
    ),
    out_shape=jax.ShapeDtypeStruct((NBLK, 1, BLK), jnp.float32),
)


@functools.partial(
    pl.kernel,
    mesh=_mesh,
    compiler_params=pltpu.CompilerParams(needs_layout_passes=False),
    out_type=jax.ShapeDtypeStruct((B,), jnp.float32),
    scratch_types=[
        pltpu.VMEM((PADB,), jnp.float32),
        pltpu.VMEM((CHUNK,), jnp.int32),
        pltpu.VMEM((CHUNK,), jnp.float32),
    ],
)
def _unpermute(pred_hbm, pos_hbm, out_hbm, pred_v, pos_v, out_v):
    cid = lax.axis_index("c")
    sid = lax.axis_index("s")
    wid = sid * NC + cid
    base = wid * CHUNK
    pltpu.sync_copy(pred_hbm, pred_v)
    pltpu.sync_copy(pos_hbm.at[pl.ds(base, CHUNK)], pos_v)
    for g in range(NGRP):
        idx = pos_v[pl.ds(g * 16, 16)]
        out_v[pl.ds(g * 16, 16)] = plsc.load_gather(pred_v, [idx])
    pltpu.sync_copy(out_v, out_hbm.at[pl.ds(base, CHUNK)])


@jax.jit
def kernel(features, affinity_type_idx, W1, b1, W2, b2):
    aff = affinity_type_idx.astype(jnp.int32)
    sorted_feat, pos, bh = _route(features, aff)
    pred = _mlp(bh, b2[:, 0], sorted_feat, W1,
                b1[:, None, :], jnp.swapaxes(W2, 1, 2))
    return _unpermute(pred.reshape(PADB), pos)
